# Initial kernel scaffold; baseline (speedup 1.0000x reference)
import jax
import jax.numpy as jnp
from jax.experimental import pallas as pl


def kernel(edge_attr, edge_index, batch, node_emb, w1, b1, w2, b2, w3, b3, root, bias, w4, b4, w5, b5):
    raise NotImplementedError("write your pallas kernel here")


# speedup vs baseline: 10.7767x; 10.7767x over previous
import functools

import jax
import jax.numpy as jnp
from jax import lax
from jax.experimental import pallas as pl
from jax.experimental.pallas import tpu as pltpu
from jax.experimental.pallas import tpu_sc as plsc

N = 10000
E = 160000
EDGE_DIM = 16
HID = 16
OUT_DIM = 10
G = 64

PACK = 8
EP = E // PACK
BE = 2000

NTILES = 32
EPT = E // NTILES
SCHUNK = 128
NCHUNK = EPT // SCHUNK
TAIL = EPT - NCHUNK * SCHUNK
ZTILES = 10
ZCH = N // ZTILES

BN = 2000


def _edge_mlp_body(ea_ref, w1_ref, b1_ref, w2_ref, b2_ref, w3_ref, b3_ref,
                   out_ref):
    h = jax.nn.relu(
        jnp.dot(ea_ref[...], w1_ref[...], preferred_element_type=jnp.float32)
        + b1_ref[...])
    h = jax.nn.relu(
        jnp.dot(h, w2_ref[...], preferred_element_type=jnp.float32)
        + b2_ref[...])
    out_ref[...] = (
        jnp.dot(h, w3_ref[...], preferred_element_type=jnp.float32)
        + b3_ref[...])


def _edge_mlp(ea8, w1b, b1b, w2b, b2b, w3b, b3b):
    wspec = pl.BlockSpec((128, 128), lambda i: (0, 0))
    bspec = pl.BlockSpec((1, 128), lambda i: (0, 0))
    return pl.pallas_call(
        _edge_mlp_body,
        grid=(EP // BE,),
        in_specs=[
            pl.BlockSpec((BE, 128), lambda i: (i, 0)),
            wspec, bspec, wspec, bspec, wspec, bspec,
        ],
        out_specs=pl.BlockSpec((BE, 128), lambda i: (i, 0)),
        out_shape=jax.ShapeDtypeStruct((EP, 128), jnp.float32),
    )(ea8, w1b, b1b, w2b, b2b, w3b, b3b)


def _scatter_body(msg_hbm, dst_hbm, tail_hbm, zeros_hbm, out_hbm,
                  msg_v, dst_v, tail_v, acc):
    c = lax.axis_index("c")
    s = lax.axis_index("s")
    wid = c * 16 + s

    @pl.when(s < ZTILES)
    def _zero():
        pltpu.sync_copy(zeros_hbm.at[pl.ds(s * ZCH, ZCH)],
                        acc.at[pl.ds(s * ZCH, ZCH)])
    plsc.subcore_barrier()

    pltpu.sync_copy(msg_hbm.at[pl.ds(wid * EPT, EPT)], msg_v)
    pltpu.sync_copy(dst_hbm.at[wid], dst_v)
    pltpu.sync_copy(tail_hbm.at[wid], tail_v)

    def chunk(j, _):
        pltpu.sync_copy(msg_v.at[pl.ds(j * SCHUNK, SCHUNK)],
                        acc.at[dst_v.at[j]], add=True)
        return 0
    lax.fori_loop(0, NCHUNK, chunk, 0)
    pltpu.sync_copy(msg_v.at[pl.ds(NCHUNK * SCHUNK, TAIL)],
                    acc.at[tail_v], add=True)
    plsc.subcore_barrier()

    @pl.when(s < ZTILES)
    def _out():
        pltpu.sync_copy(acc.at[pl.ds(s * ZCH, ZCH)],
                        out_hbm.at[c, pl.ds(s * ZCH, ZCH)])


@functools.cache
def _build_scatter_add():
    return pl.kernel(
        _scatter_body,
        out_type=jax.ShapeDtypeStruct((2, N, HID), jnp.float32),
        mesh=plsc.VectorSubcoreMesh(core_axis_name="c", subcore_axis_name="s",
                                    num_cores=2, num_subcores=16),
        compiler_params=pltpu.CompilerParams(use_tc_tiling_on_sc=False),
        scratch_types=[
            pltpu.VMEM((EPT, HID), jnp.float32),
            pltpu.VMEM((NCHUNK, SCHUNK), jnp.int32),
            pltpu.VMEM((TAIL,), jnp.int32),
            pltpu.VMEM_SHARED((N, HID), jnp.float32),
        ],
    )


def _pool_head_body(part_ref, ids_ref, ne_ref, root_ref, bias_ref,
                    w4_ref, b4_ref, w5_ref, b5_ref, out_ref, sums, counts):
    i = pl.program_id(0)

    @pl.when(i == 0)
    def _init():
        sums[...] = jnp.zeros_like(sums)
        counts[...] = jnp.zeros_like(counts)

    c16 = (jnp.dot(ne_ref[...], root_ref[...],
                   preferred_element_type=jnp.float32) + bias_ref[...])
    x = jax.nn.relu(part_ref[0] + part_ref[1] + c16)
    ids = ids_ref[0, 0, :]
    onehot = (lax.broadcasted_iota(jnp.int32, (G, BN), 0)
              == ids[None, :]).astype(jnp.float32)
    sums[...] += jnp.dot(onehot, x, preferred_element_type=jnp.float32)
    counts[...] += jnp.sum(onehot, axis=1, keepdims=True)

    @pl.when(i == pl.num_programs(0) - 1)
    def _final():
        pooled = sums[...] / jnp.maximum(counts[...], 1.0)
        h = jax.nn.relu(
            jnp.dot(pooled, w4_ref[...], preferred_element_type=jnp.float32)
            + b4_ref[...])
        out_ref[...] = (
            jnp.dot(h, w5_ref[...], preferred_element_type=jnp.float32)
            + b5_ref[...])


def _pool_head(partials, ids3, ne, root, bias2, w4, b42, w5, b52):
    return pl.pallas_call(
        _pool_head_body,
        grid=(N // BN,),
        in_specs=[
            pl.BlockSpec((2, BN, HID), lambda i: (0, i, 0)),
            pl.BlockSpec((1, 1, BN), lambda i: (i, 0, 0)),
            pl.BlockSpec((1, HID), lambda i: (0, 0)),
            pl.BlockSpec((HID, HID), lambda i: (0, 0)),
            pl.BlockSpec((1, HID), lambda i: (0, 0)),
            pl.BlockSpec((HID, 2 * HID), lambda i: (0, 0)),
            pl.BlockSpec((1, 2 * HID), lambda i: (0, 0)),
            pl.BlockSpec((2 * HID, OUT_DIM), lambda i: (0, 0)),
            pl.BlockSpec((1, OUT_DIM), lambda i: (0, 0)),
        ],
        out_specs=pl.BlockSpec((G, OUT_DIM), lambda i: (0, 0)),
        out_shape=jax.ShapeDtypeStruct((G, OUT_DIM), jnp.float32),
        scratch_shapes=[
            pltpu.VMEM((G, HID), jnp.float32),
            pltpu.VMEM((G, 1), jnp.float32),
        ],
    )(partials, ids3, ne, root, bias2, w4, b42, w5, b52)


def kernel(edge_attr, edge_index, batch, node_emb, w1, b1, w2, b2, w3, b3,
           root, bias, w4, b4, w5, b5):
    ne = node_emb[0]

    w3f = jnp.einsum("i,kio->ko", ne, w3.reshape(HID, EDGE_DIM, HID))
    b3f = ne @ b3.reshape(EDGE_DIM, HID)

    eye8 = jnp.eye(PACK, dtype=jnp.float32)
    w1b = jnp.kron(eye8, w1)
    w2b = jnp.kron(eye8, w2)
    w3b = jnp.kron(eye8, w3f)
    b1b = jnp.tile(b1, PACK)[None, :]
    b2b = jnp.tile(b2, PACK)[None, :]
    b3b = jnp.tile(b3f, PACK)[None, :]

    ea8 = edge_attr.reshape(EP, 128)
    msg8 = _edge_mlp(ea8, w1b, b1b, w2b, b2b, w3b, b3b)
    msg = msg8.reshape(E, HID)

    dst2 = edge_index[1].reshape(NTILES, EPT)
    dst_main = dst2[:, :NCHUNK * SCHUNK].reshape(NTILES, NCHUNK, SCHUNK)
    dst_tail = dst2[:, NCHUNK * SCHUNK:]
    zeros = jnp.zeros((N, HID), jnp.float32)
    partials = _build_scatter_add()(msg, dst_main, dst_tail, zeros)

    ids3 = batch.reshape(N // BN, 1, BN)
    return _pool_head(partials, ids3, node_emb, root, bias[None, :],
                      w4, b4[None, :], w5, b5[None, :])


# speedup vs baseline: 17.7047x; 1.6429x over previous
import functools

import jax
import jax.numpy as jnp
from jax import lax
from jax.experimental import pallas as pl
from jax.experimental.pallas import tpu as pltpu
from jax.experimental.pallas import tpu_sc as plsc

N = 10000
E = 160000
EDGE_DIM = 16
HID = 16
OUT_DIM = 10
G = 64

PACK = 8
SUB = 2000
BE = PACK * SUB
NBLK = E // BE
EP = E
EPK = EP // PACK

NTILES = 32
EPT = EP // NTILES
SCHUNK = 128
NCHUNK = EPT // SCHUNK
TAIL = EPT - NCHUNK * SCHUNK
NPAD = 10240
NPT = NPAD // 16

NPK = NPAD // PACK
BNP = NPK // 5


def _edge_mlp_body(eat_ref, w1_ref, b1_ref, w2_ref, b2_ref, w3_ref, b3_ref,
                   out_ref):
    x = eat_ref[...]
    lhs = jnp.concatenate(
        [x[:, a * SUB:(a + 1) * SUB] for a in range(PACK)], axis=0)
    h = jax.nn.relu(
        lax.dot_general(lhs, w1_ref[...], (((0,), (0,)), ((), ())),
                        preferred_element_type=jnp.float32) + b1_ref[...])
    h = jax.nn.relu(
        jnp.dot(h, w2_ref[...], preferred_element_type=jnp.float32)
        + b2_ref[...])
    out_ref[...] = (
        jnp.dot(h, w3_ref[...], preferred_element_type=jnp.float32)
        + b3_ref[...])


def _edge_mlp(eat, w1b, b1b, w2b, b2b, w3b, b3b):
    wspec = pl.BlockSpec((128, 128), lambda i: (0, 0))
    bspec = pl.BlockSpec((1, 128), lambda i: (0, 0))
    return pl.pallas_call(
        _edge_mlp_body,
        grid=(NBLK,),
        in_specs=[
            pl.BlockSpec((EDGE_DIM, BE), lambda i: (0, i)),
            wspec, bspec, wspec, bspec, wspec, bspec,
        ],
        out_specs=pl.BlockSpec((SUB, 128), lambda i: (i, 0)),
        out_shape=jax.ShapeDtypeStruct((EPK, 128), jnp.float32),
    )(eat, w1b, b1b, w2b, b2b, w3b, b3b)


def _scatter_body(msg_hbm, dst_hbm, tail_hbm, zeros_hbm, out_hbm,
                  msg_v, dst_v, tail_v, acc):
    c = lax.axis_index("c")
    s = lax.axis_index("s")
    wid = c * 16 + s

    pltpu.sync_copy(zeros_hbm.at[pl.ds(s * NPT, NPT)],
                    acc.at[pl.ds(s * NPT, NPT)])
    plsc.subcore_barrier()

    pltpu.sync_copy(msg_hbm.at[pl.ds(wid * EPT, EPT)], msg_v)
    pltpu.sync_copy(dst_hbm.at[wid], dst_v)
    pltpu.sync_copy(tail_hbm.at[wid], tail_v)

    def chunk(j, _):
        pltpu.sync_copy(msg_v.at[pl.ds(j * SCHUNK, SCHUNK)],
                        acc.at[dst_v.at[j]], add=True)
        return 0
    lax.fori_loop(0, NCHUNK, chunk, 0)
    pltpu.sync_copy(msg_v.at[pl.ds(NCHUNK * SCHUNK, TAIL)],
                    acc.at[tail_v], add=True)
    plsc.subcore_barrier()

    pltpu.sync_copy(acc.at[pl.ds(s * NPT, NPT)],
                    out_hbm.at[c, pl.ds(s * NPT, NPT)])


@functools.cache
def _build_scatter_add():
    return pl.kernel(
        _scatter_body,
        out_type=jax.ShapeDtypeStruct((2, NPAD, HID), jnp.float32),
        mesh=plsc.VectorSubcoreMesh(core_axis_name="c", subcore_axis_name="s",
                                    num_cores=2, num_subcores=16),
        scratch_types=[
            pltpu.VMEM((EPT, HID), jnp.float32),
            pltpu.VMEM((NCHUNK, SCHUNK), jnp.int32),
            pltpu.VMEM((TAIL,), jnp.int32),
            pltpu.VMEM_SHARED((NPAD, HID), jnp.float32),
        ],
        compiler_params=pltpu.CompilerParams(use_tc_tiling_on_sc=False),
    )


def _pool_head_body(part_ref, ids_ref, ne_ref, root_ref, bias_ref,
                    w4_ref, b4_ref, w5_ref, b5_ref, out_ref, sums, counts):
    i = pl.program_id(0)

    @pl.when(i == 0)
    def _init():
        sums[...] = jnp.zeros_like(sums)
        counts[...] = jnp.zeros_like(counts)

    c16 = (jnp.dot(ne_ref[...], root_ref[...],
                   preferred_element_type=jnp.float32) + bias_ref[...])
    c128 = jnp.concatenate([c16] * PACK, axis=1)
    x8 = jax.nn.relu(part_ref[0] + part_ref[1] + c128)
    for a in range(PACK):
        ids = ids_ref[a, :]
        onehot = (lax.broadcasted_iota(jnp.int32, (G, BNP), 0)
                  == ids[None, :]).astype(jnp.float32)
        xa = x8[:, a * HID:(a + 1) * HID]
        sums[...] += jnp.dot(onehot, xa,
                             preferred_element_type=jnp.float32)
        counts[...] += jnp.sum(onehot, axis=1, keepdims=True)

    @pl.when(i == pl.num_programs(0) - 1)
    def _final():
        pooled = sums[...] / jnp.maximum(counts[...], 1.0)
        h = jax.nn.relu(
            jnp.dot(pooled, w4_ref[...], preferred_element_type=jnp.float32)
            + b4_ref[...])
        out_ref[...] = (
            jnp.dot(h, w5_ref[...], preferred_element_type=jnp.float32)
            + b5_ref[...])


def _pool_head(partials, ids8, ne, root, bias2, w4, b42, w5, b52):
    return pl.pallas_call(
        _pool_head_body,
        grid=(NPK // BNP,),
        in_specs=[
            pl.BlockSpec((2, BNP, 128), lambda i: (0, i, 0)),
            pl.BlockSpec((PACK, BNP), lambda i: (0, i)),
            pl.BlockSpec((1, HID), lambda i: (0, 0)),
            pl.BlockSpec((HID, HID), lambda i: (0, 0)),
            pl.BlockSpec((1, HID), lambda i: (0, 0)),
            pl.BlockSpec((HID, 2 * HID), lambda i: (0, 0)),
            pl.BlockSpec((1, 2 * HID), lambda i: (0, 0)),
            pl.BlockSpec((2 * HID, OUT_DIM), lambda i: (0, 0)),
            pl.BlockSpec((1, OUT_DIM), lambda i: (0, 0)),
        ],
        out_specs=pl.BlockSpec((G, OUT_DIM), lambda i: (0, 0)),
        out_shape=jax.ShapeDtypeStruct((G, OUT_DIM), jnp.float32),
        scratch_shapes=[
            pltpu.VMEM((G, HID), jnp.float32),
            pltpu.VMEM((G, 1), jnp.float32),
        ],
    )(partials, ids8, ne, root, bias2, w4, b42, w5, b52)


def kernel(edge_attr, edge_index, batch, node_emb, w1, b1, w2, b2, w3, b3,
           root, bias, w4, b4, w5, b5):
    ne = node_emb[0]

    w3f = jnp.einsum("i,kio->ko", ne, w3.reshape(HID, EDGE_DIM, HID))
    b3f = ne @ b3.reshape(EDGE_DIM, HID)

    eye8 = jnp.eye(PACK, dtype=jnp.float32)
    w1b = jnp.kron(eye8, w1)
    w2b = jnp.kron(eye8, w2)
    w3b = jnp.kron(eye8, w3f)
    b1b = jnp.tile(b1, PACK)[None, :]
    b2b = jnp.tile(b2, PACK)[None, :]
    b3b = jnp.tile(b3f, PACK)[None, :]

    eat = edge_attr.T
    msg8 = _edge_mlp(eat, w1b, b1b, w2b, b2b, w3b, b3b)
    msg = msg8.reshape(EP, HID)

    dst = edge_index[1]
    dst_perm = (
        dst.reshape(NBLK, PACK, SUB).transpose(0, 2, 1).reshape(NTILES, EPT))
    dst3 = dst_perm[:, :NCHUNK * SCHUNK].reshape(NTILES, NCHUNK, SCHUNK)
    dst_tail = dst_perm[:, NCHUNK * SCHUNK:]
    zeros = jnp.zeros((NPAD, HID), jnp.float32)
    partials = _build_scatter_add()(msg, dst3, dst_tail, zeros)

    ids8 = jnp.concatenate(
        [batch, jnp.full((NPAD - N,), G, dtype=jnp.int32)]
    ).reshape(NPK, PACK).T
    part8 = partials.reshape(2, NPK, 128)
    return _pool_head(part8, ids8, node_emb, root, bias[None, :],
                      w4, b4[None, :], w5, b5[None, :])


# speedup vs baseline: 20.2396x; 1.1432x over previous
import functools

import jax
import jax.numpy as jnp
from jax import lax
from jax.experimental import pallas as pl
from jax.experimental.pallas import tpu as pltpu
from jax.experimental.pallas import tpu_sc as plsc

N = 10000
E = 160000
EDGE_DIM = 16
HID = 16
OUT_DIM = 10
G = 64

PACK = 8
SUB = 10000
BE = PACK * SUB
NBLK = E // BE
EP = E
EPK = EP // PACK

NTILES = 32
EPT = EP // NTILES
RPT = EPT // PACK
TPB = NTILES // NBLK
SCHUNK = 128
NCHUNK = EPT // SCHUNK
TAIL = EPT - NCHUNK * SCHUNK
SEG = 648
NPAD = 10240
NPT = NPAD // 16

NPK = NPAD // PACK
BNP = NPK // 5


def _edge_mlp_body(eat_ref, w1_ref, b1_ref, w2_ref, b2_ref, w3_ref, b3_ref,
                   out_ref):
    x = eat_ref[...]
    lhs = jnp.concatenate(
        [x[:, a * SUB:(a + 1) * SUB] for a in range(PACK)], axis=0)
    h = jax.nn.relu(
        lax.dot_general(lhs, w1_ref[...], (((0,), (0,)), ((), ())),
                        preferred_element_type=jnp.float32) + b1_ref[...])
    h = jax.nn.relu(
        jnp.dot(h, w2_ref[...], preferred_element_type=jnp.float32)
        + b2_ref[...])
    out_ref[...] = (
        jnp.dot(h, w3_ref[...], preferred_element_type=jnp.float32)
        + b3_ref[...])


def _edge_mlp(eat, w1b, b1b, w2b, b2b, w3b, b3b):
    wspec = pl.BlockSpec((128, 128), lambda i: (0, 0))
    bspec = pl.BlockSpec((1, 128), lambda i: (0, 0))
    return pl.pallas_call(
        _edge_mlp_body,
        grid=(NBLK,),
        in_specs=[
            pl.BlockSpec((EDGE_DIM, BE), lambda i: (0, i)),
            wspec, bspec, wspec, bspec, wspec, bspec,
        ],
        out_specs=pl.BlockSpec((SUB, 128), lambda i: (i, 0)),
        out_shape=jax.ShapeDtypeStruct((EPK, 128), jnp.float32),
    )(eat, w1b, b1b, w2b, b2b, w3b, b3b)


def _scatter_body(msg_hbm, dst_hbm, zeros_hbm, out_hbm,
                  msg_v, stage_v, dst_v, acc):
    c = lax.axis_index("c")
    s = lax.axis_index("s")
    wid = c * 16 + s
    blk = wid // TPB
    tloc = wid % TPB
    shift = (RPT * tloc) % 8

    pltpu.sync_copy(zeros_hbm.at[pl.ds(s * NPT, NPT)],
                    acc.at[pl.ds(s * NPT, NPT)])
    plsc.subcore_barrier()

    pltpu.sync_copy(msg_hbm.at[pl.ds(wid * EPT, EPT)], msg_v)

    for a in range(PACK):
        base8 = (BE // 8) * blk + (SUB // 8) * a + (RPT * tloc) // 8
        pltpu.sync_copy(dst_hbm.at[pl.ds(8 * base8, SEG)], stage_v.at[a])

    io = lax.iota(jnp.int32, 16)
    for a in range(PACK):
        row = jnp.full((16,), a, jnp.int32)
        def reorder(g, _):
            rho = 16 * g + io
            mask = rho < RPT
            vals = plsc.load_gather(stage_v, [row, shift + rho], mask=mask)
            m = 8 * rho + a
            plsc.store_scatter(dst_v, [m >> 7, m & (SCHUNK - 1)], vals,
                               mask=mask)
            return 0
        lax.fori_loop(0, 40, reorder, 0)

    def chunk(j, _):
        pltpu.sync_copy(msg_v.at[pl.ds(j * SCHUNK, SCHUNK)],
                        acc.at[dst_v.at[j]], add=True)
        return 0
    lax.fori_loop(0, NCHUNK, chunk, 0)
    pltpu.sync_copy(msg_v.at[pl.ds(NCHUNK * SCHUNK, TAIL)],
                    acc.at[dst_v.at[NCHUNK, pl.ds(0, TAIL)]], add=True)
    plsc.subcore_barrier()

    pltpu.sync_copy(acc.at[pl.ds(s * NPT, NPT)],
                    out_hbm.at[c, pl.ds(s * NPT, NPT)])


@functools.cache
def _build_scatter_add():
    return pl.kernel(
        _scatter_body,
        out_type=jax.ShapeDtypeStruct((2, NPAD, HID), jnp.float32),
        mesh=plsc.VectorSubcoreMesh(core_axis_name="c", subcore_axis_name="s",
                                    num_cores=2, num_subcores=16),
        scratch_types=[
            pltpu.VMEM((EPT, HID), jnp.float32),
            pltpu.VMEM((PACK, SEG), jnp.int32),
            pltpu.VMEM((NCHUNK + 1, SCHUNK), jnp.int32),
            pltpu.VMEM_SHARED((NPAD, HID), jnp.float32),
        ],
        compiler_params=pltpu.CompilerParams(
            use_tc_tiling_on_sc=False, needs_layout_passes=False),
    )


def _pool_head_body(part_ref, ids_ref, ne_ref, root_ref, bias_ref,
                    w4_ref, b4_ref, w5_ref, b5_ref, out_ref, sums, counts):
    i = pl.program_id(0)

    @pl.when(i == 0)
    def _init():
        sums[...] = jnp.zeros_like(sums)
        counts[...] = jnp.zeros_like(counts)

    c16 = (jnp.dot(ne_ref[...], root_ref[...],
                   preferred_element_type=jnp.float32) + bias_ref[...])
    c128 = jnp.concatenate([c16] * PACK, axis=1)
    x8 = jax.nn.relu(part_ref[0] + part_ref[1] + c128)
    for a in range(PACK):
        ids = ids_ref[a, :]
        onehot = (lax.broadcasted_iota(jnp.int32, (G, BNP), 0)
                  == ids[None, :]).astype(jnp.float32)
        xa = x8[:, a * HID:(a + 1) * HID]
        sums[...] += jnp.dot(onehot, xa,
                             preferred_element_type=jnp.float32)
        counts[...] += jnp.sum(onehot, axis=1, keepdims=True)

    @pl.when(i == pl.num_programs(0) - 1)
    def _final():
        pooled = sums[...] / jnp.maximum(counts[...], 1.0)
        h = jax.nn.relu(
            jnp.dot(pooled, w4_ref[...], preferred_element_type=jnp.float32)
            + b4_ref[...])
        out_ref[...] = (
            jnp.dot(h, w5_ref[...], preferred_element_type=jnp.float32)
            + b5_ref[...])


def _pool_head(partials, ids8, ne, root, bias2, w4, b42, w5, b52):
    return pl.pallas_call(
        _pool_head_body,
        grid=(NPK // BNP,),
        in_specs=[
            pl.BlockSpec((2, BNP, 128), lambda i: (0, i, 0)),
            pl.BlockSpec((PACK, BNP), lambda i: (0, i)),
            pl.BlockSpec((1, HID), lambda i: (0, 0)),
            pl.BlockSpec((HID, HID), lambda i: (0, 0)),
            pl.BlockSpec((1, HID), lambda i: (0, 0)),
            pl.BlockSpec((HID, 2 * HID), lambda i: (0, 0)),
            pl.BlockSpec((1, 2 * HID), lambda i: (0, 0)),
            pl.BlockSpec((2 * HID, OUT_DIM), lambda i: (0, 0)),
            pl.BlockSpec((1, OUT_DIM), lambda i: (0, 0)),
        ],
        out_specs=pl.BlockSpec((G, OUT_DIM), lambda i: (0, 0)),
        out_shape=jax.ShapeDtypeStruct((G, OUT_DIM), jnp.float32),
        scratch_shapes=[
            pltpu.VMEM((G, HID), jnp.float32),
            pltpu.VMEM((G, 1), jnp.float32),
        ],
    )(partials, ids8, ne, root, bias2, w4, b42, w5, b52)


def kernel(edge_attr, edge_index, batch, node_emb, w1, b1, w2, b2, w3, b3,
           root, bias, w4, b4, w5, b5):
    ne = node_emb[0]

    w3f = jnp.einsum("i,kio->ko", ne, w3.reshape(HID, EDGE_DIM, HID))
    b3f = ne @ b3.reshape(EDGE_DIM, HID)

    eye8 = jnp.eye(PACK, dtype=jnp.float32)
    w1b = jnp.kron(eye8, w1)
    w2b = jnp.kron(eye8, w2)
    w3b = jnp.kron(eye8, w3f)
    b1b = jnp.tile(b1, PACK)[None, :]
    b2b = jnp.tile(b2, PACK)[None, :]
    b3b = jnp.tile(b3f, PACK)[None, :]

    eat = edge_attr.T
    msg8 = _edge_mlp(eat, w1b, b1b, w2b, b2b, w3b, b3b)
    msg = msg8.reshape(EP, HID)

    dst1 = jnp.concatenate(
        [edge_index[1], jnp.zeros((16,), dtype=jnp.int32)])
    zeros = jnp.zeros((NPAD, HID), jnp.float32)
    partials = _build_scatter_add()(msg, dst1, zeros)

    ids8 = jnp.concatenate(
        [batch, jnp.full((NPAD - N,), G, dtype=jnp.int32)]
    ).reshape(NPK, PACK).T
    part8 = partials.reshape(2, NPK, 128)
    return _pool_head(part8, ids8, node_emb, root, bias[None, :],
                      w4, b4[None, :], w5, b5[None, :])


# speedup vs baseline: 23.0677x; 1.1397x over previous
import functools

import jax
import jax.numpy as jnp
from jax import lax
from jax.experimental import pallas as pl
from jax.experimental.pallas import tpu as pltpu
from jax.experimental.pallas import tpu_sc as plsc

N = 10000
E = 160000
EDGE_DIM = 16
HID = 16
OUT_DIM = 10
G = 64

PACK = 8
SUB = 10000
BE = PACK * SUB
NBLK = E // BE
EP = E
EPK = EP // PACK

NTILES = 32
EPT = EP // NTILES
RPT = EPT // PACK
TPB = NTILES // NBLK
SCHUNK = 128
NCHUNK = EPT // SCHUNK
TAIL = EPT - NCHUNK * SCHUNK
SEG = 648
NPAD = 10240
NPT = NPAD // 16

NPK = NPAD // PACK
BNP = NPK // 5


def _edge_mlp_body(eat_ref, w1_ref, b1_ref, w2_ref, b2_ref, w3_ref, b3_ref,
                   out_ref):
    x = eat_ref[...]
    lhs = jnp.concatenate(
        [x[:, a * SUB:(a + 1) * SUB] for a in range(PACK)], axis=0)
    h = jax.nn.relu(
        lax.dot_general(lhs, w1_ref[...], (((0,), (0,)), ((), ())),
                        preferred_element_type=jnp.float32) + b1_ref[...])
    h = jax.nn.relu(
        jnp.dot(h, w2_ref[...], preferred_element_type=jnp.float32)
        + b2_ref[...])
    out_ref[...] = (
        jnp.dot(h, w3_ref[...], preferred_element_type=jnp.float32)
        + b3_ref[...])


def _edge_mlp(eat, w1b, b1b, w2b, b2b, w3b, b3b):
    wspec = pl.BlockSpec((128, 128), lambda i: (0, 0))
    bspec = pl.BlockSpec((1, 128), lambda i: (0, 0))
    return pl.pallas_call(
        _edge_mlp_body,
        grid=(NBLK,),
        in_specs=[
            pl.BlockSpec((EDGE_DIM, BE), lambda i: (0, i)),
            wspec, bspec, wspec, bspec, wspec, bspec,
        ],
        out_specs=pl.BlockSpec((SUB, 128), lambda i: (i, 0)),
        out_shape=jax.ShapeDtypeStruct((EPK, 128), jnp.float32),
    )(eat, w1b, b1b, w2b, b2b, w3b, b3b)


def _scatter_body(msg_hbm, dst_hbm, zeros_hbm, out_hbm,
                  msg_v, stage_v, dst_v, acc, sem_m, sem_d, sem_s):
    c = lax.axis_index("c")
    s = lax.axis_index("s")
    wid = c * 16 + s
    blk = wid // TPB
    tloc = wid % TPB
    shift = (RPT * tloc) % 8

    msg_cp = pltpu.async_copy(msg_hbm.at[pl.ds(wid * EPT, EPT)], msg_v,
                              sem_m)

    dst_cps = []
    for a in range(PACK):
        base8 = (BE // 8) * blk + (SUB // 8) * a + (RPT * tloc) // 8
        dst_cps.append(pltpu.async_copy(
            dst_hbm.at[pl.ds(8 * base8, SEG)], stage_v.at[a], sem_d))

    pltpu.sync_copy(zeros_hbm.at[pl.ds(s * NPT, NPT)],
                    acc.at[pl.ds(s * NPT, NPT)])
    plsc.subcore_barrier()
    for cp in dst_cps:
        cp.wait()

    io = lax.iota(jnp.int32, 16)
    for a in range(PACK):
        row = jnp.full((16,), a, jnp.int32)
        def reorder(g, _):
            rho = 16 * g + io
            mask = rho < RPT
            vals = plsc.load_gather(stage_v, [row, shift + rho], mask=mask)
            m = 8 * rho + a
            plsc.store_scatter(dst_v, [m >> 7, m & (SCHUNK - 1)], vals,
                               mask=mask)
            return 0
        lax.fori_loop(0, 40, reorder, 0)

    msg_cp.wait()
    cps = [pltpu.async_copy(msg_v.at[pl.ds(j * SCHUNK, SCHUNK)],
                            acc.at[dst_v.at[j]], sem_s, add=True)
           for j in range(NCHUNK)]
    cps.append(pltpu.async_copy(
        msg_v.at[pl.ds(NCHUNK * SCHUNK, TAIL)],
        acc.at[dst_v.at[NCHUNK, pl.ds(0, TAIL)]], sem_s, add=True))
    for cp in cps:
        cp.wait()
    plsc.subcore_barrier()

    pltpu.sync_copy(acc.at[pl.ds(s * NPT, NPT)],
                    out_hbm.at[c, pl.ds(s * NPT, NPT)])


@functools.cache
def _build_scatter_add():
    return pl.kernel(
        _scatter_body,
        out_type=jax.ShapeDtypeStruct((2, NPAD, HID), jnp.float32),
        mesh=plsc.VectorSubcoreMesh(core_axis_name="c", subcore_axis_name="s",
                                    num_cores=2, num_subcores=16),
        scratch_types=[
            pltpu.VMEM((EPT, HID), jnp.float32),
            pltpu.VMEM((PACK, SEG), jnp.int32),
            pltpu.VMEM((NCHUNK + 1, SCHUNK), jnp.int32),
            pltpu.VMEM_SHARED((NPAD, HID), jnp.float32),
            pltpu.SemaphoreType.DMA,
            pltpu.SemaphoreType.DMA,
            pltpu.SemaphoreType.DMA,
        ],
        compiler_params=pltpu.CompilerParams(
            use_tc_tiling_on_sc=False, needs_layout_passes=False),
    )


def _pool_head_body(part_ref, ids_ref, ne_ref, root_ref, bias_ref,
                    w4_ref, b4_ref, w5_ref, b5_ref, out_ref, sums, counts):
    i = pl.program_id(0)

    @pl.when(i == 0)
    def _init():
        sums[...] = jnp.zeros_like(sums)
        counts[...] = jnp.zeros_like(counts)

    c16 = (jnp.dot(ne_ref[...], root_ref[...],
                   preferred_element_type=jnp.float32) + bias_ref[...])
    c128 = jnp.concatenate([c16] * PACK, axis=1)
    x8 = jax.nn.relu(part_ref[0] + part_ref[1] + c128)
    for a in range(PACK):
        ids = ids_ref[a, :]
        onehot = (lax.broadcasted_iota(jnp.int32, (G, BNP), 0)
                  == ids[None, :]).astype(jnp.float32)
        xa = x8[:, a * HID:(a + 1) * HID]
        sums[...] += jnp.dot(onehot, xa,
                             preferred_element_type=jnp.float32)
        counts[...] += jnp.sum(onehot, axis=1, keepdims=True)

    @pl.when(i == pl.num_programs(0) - 1)
    def _final():
        pooled = sums[...] / jnp.maximum(counts[...], 1.0)
        h = jax.nn.relu(
            jnp.dot(pooled, w4_ref[...], preferred_element_type=jnp.float32)
            + b4_ref[...])
        out_ref[...] = (
            jnp.dot(h, w5_ref[...], preferred_element_type=jnp.float32)
            + b5_ref[...])


def _pool_head(partials, ids8, ne, root, bias2, w4, b42, w5, b52):
    return pl.pallas_call(
        _pool_head_body,
        grid=(NPK // BNP,),
        in_specs=[
            pl.BlockSpec((2, BNP, 128), lambda i: (0, i, 0)),
            pl.BlockSpec((PACK, BNP), lambda i: (0, i)),
            pl.BlockSpec((1, HID), lambda i: (0, 0)),
            pl.BlockSpec((HID, HID), lambda i: (0, 0)),
            pl.BlockSpec((1, HID), lambda i: (0, 0)),
            pl.BlockSpec((HID, 2 * HID), lambda i: (0, 0)),
            pl.BlockSpec((1, 2 * HID), lambda i: (0, 0)),
            pl.BlockSpec((2 * HID, OUT_DIM), lambda i: (0, 0)),
            pl.BlockSpec((1, OUT_DIM), lambda i: (0, 0)),
        ],
        out_specs=pl.BlockSpec((G, OUT_DIM), lambda i: (0, 0)),
        out_shape=jax.ShapeDtypeStruct((G, OUT_DIM), jnp.float32),
        scratch_shapes=[
            pltpu.VMEM((G, HID), jnp.float32),
            pltpu.VMEM((G, 1), jnp.float32),
        ],
    )(partials, ids8, ne, root, bias2, w4, b42, w5, b52)


def kernel(edge_attr, edge_index, batch, node_emb, w1, b1, w2, b2, w3, b3,
           root, bias, w4, b4, w5, b5):
    ne = node_emb[0]

    w3f = jnp.einsum("i,kio->ko", ne, w3.reshape(HID, EDGE_DIM, HID))
    b3f = ne @ b3.reshape(EDGE_DIM, HID)

    eye8 = jnp.eye(PACK, dtype=jnp.float32)
    w1b = jnp.kron(eye8, w1)
    w2b = jnp.kron(eye8, w2)
    w3b = jnp.kron(eye8, w3f)
    b1b = jnp.tile(b1, PACK)[None, :]
    b2b = jnp.tile(b2, PACK)[None, :]
    b3b = jnp.tile(b3f, PACK)[None, :]

    eat = edge_attr.T
    msg8 = _edge_mlp(eat, w1b, b1b, w2b, b2b, w3b, b3b)
    msg = msg8.reshape(EP, HID)

    dst1 = jnp.concatenate(
        [edge_index[1], jnp.zeros((16,), dtype=jnp.int32)])
    zeros = jnp.zeros((NPAD, HID), jnp.float32)
    partials = _build_scatter_add()(msg, dst1, zeros)

    ids8 = jnp.concatenate(
        [batch, jnp.full((NPAD - N,), G, dtype=jnp.int32)]
    ).reshape(NPK, PACK).T
    part8 = partials.reshape(2, NPK, 128)
    return _pool_head(part8, ids8, node_emb, root, bias[None, :],
                      w4, b4[None, :], w5, b5[None, :])


# speedup vs baseline: 23.3292x; 1.0113x over previous
import functools

import jax
import jax.numpy as jnp
from jax import lax
from jax.experimental import pallas as pl
from jax.experimental.pallas import tpu as pltpu
from jax.experimental.pallas import tpu_sc as plsc

N = 10000
E = 160000
EDGE_DIM = 16
HID = 16
OUT_DIM = 10
G = 64

PACK = 8
SUB = 10000
BE = PACK * SUB
NBLK = E // BE
EP = E
EPK = EP // PACK

NTILES = 32
EPT = EP // NTILES
RPT = EPT // PACK
TPB = NTILES // NBLK
SCHUNK = 128
NCHUNK = EPT // SCHUNK
TAIL = EPT - NCHUNK * SCHUNK
SEG = 648
NPAD = 10240
NPT = NPAD // 16

NPK = NPAD // PACK
BNP = NPK // 5


def _edge_mlp_body(eat_ref, w1_ref, b1_ref, w2_ref, b2_ref, w3_ref, b3_ref,
                   out_ref):
    rb = lax.broadcasted_iota(jnp.int32, (128, 128), 0) // HID
    cb = lax.broadcasted_iota(jnp.int32, (128, 128), 1) // HID
    bmask = (rb == cb).astype(jnp.float32)
    w1b = jnp.tile(w1_ref[...], (PACK, PACK)) * bmask
    w2b = jnp.tile(w2_ref[...], (PACK, PACK)) * bmask
    w3b = jnp.tile(w3_ref[...], (PACK, PACK)) * bmask
    b1b = jnp.tile(b1_ref[...], (1, PACK))
    b2b = jnp.tile(b2_ref[...], (1, PACK))
    b3b = jnp.tile(b3_ref[...], (1, PACK))

    x = eat_ref[...]
    lhs = jnp.concatenate(
        [x[:, a * SUB:(a + 1) * SUB] for a in range(PACK)], axis=0)
    h = jax.nn.relu(
        lax.dot_general(lhs, w1b, (((0,), (0,)), ((), ())),
                        preferred_element_type=jnp.float32) + b1b)
    h = jax.nn.relu(
        jnp.dot(h, w2b, preferred_element_type=jnp.float32) + b2b)
    out_ref[...] = (
        jnp.dot(h, w3b, preferred_element_type=jnp.float32) + b3b)


def _edge_mlp(eat, w1, b1, w2, b2, w3f, b3f):
    wspec = pl.BlockSpec((HID, HID), lambda i: (0, 0))
    bspec = pl.BlockSpec((1, HID), lambda i: (0, 0))
    return pl.pallas_call(
        _edge_mlp_body,
        grid=(NBLK,),
        in_specs=[
            pl.BlockSpec((EDGE_DIM, BE), lambda i: (0, i)),
            wspec, bspec, wspec, bspec, wspec, bspec,
        ],
        out_specs=pl.BlockSpec((SUB, 128), lambda i: (i, 0)),
        out_shape=jax.ShapeDtypeStruct((EPK, 128), jnp.float32),
    )(eat, w1, b1, w2, b2, w3f, b3f)


def _scatter_body(msg_hbm, dst_hbm, zeros_hbm, out_hbm,
                  msg_v, stage_v, dst_v, acc, sem_m, sem_d, sem_s):
    c = lax.axis_index("c")
    s = lax.axis_index("s")
    wid = c * 16 + s
    blk = wid // TPB
    tloc = wid % TPB
    shift = (RPT * tloc) % 8

    msg_cp = pltpu.async_copy(msg_hbm.at[pl.ds(wid * EPT, EPT)], msg_v,
                              sem_m)

    dst_cps = []
    for a in range(PACK):
        base8 = (BE // 8) * blk + (SUB // 8) * a + (RPT * tloc) // 8
        dst_cps.append(pltpu.async_copy(
            dst_hbm.at[pl.ds(8 * base8, SEG)], stage_v.at[a], sem_d))

    pltpu.sync_copy(zeros_hbm, acc.at[pl.ds(s * NPT, NPT)])
    plsc.subcore_barrier()
    for cp in dst_cps:
        cp.wait()

    io = lax.iota(jnp.int32, 16)
    for a in range(PACK):
        row = jnp.full((16,), a, jnp.int32)
        def reorder(g, _):
            rho = 16 * g + io
            mask = rho < RPT
            vals = plsc.load_gather(stage_v, [row, shift + rho], mask=mask)
            m = 8 * rho + a
            plsc.store_scatter(dst_v, [m >> 7, m & (SCHUNK - 1)], vals,
                               mask=mask)
            return 0
        lax.fori_loop(0, 40, reorder, 0)

    msg_cp.wait()
    cps = [pltpu.async_copy(msg_v.at[pl.ds(j * SCHUNK, SCHUNK)],
                            acc.at[dst_v.at[j]], sem_s, add=True)
           for j in range(NCHUNK)]
    cps.append(pltpu.async_copy(
        msg_v.at[pl.ds(NCHUNK * SCHUNK, TAIL)],
        acc.at[dst_v.at[NCHUNK, pl.ds(0, TAIL)]], sem_s, add=True))
    for cp in cps:
        cp.wait()
    plsc.subcore_barrier()

    pltpu.sync_copy(acc.at[pl.ds(s * NPT, NPT)],
                    out_hbm.at[c, pl.ds(s * NPT, NPT)])


@functools.cache
def _build_scatter_add():
    return pl.kernel(
        _scatter_body,
        out_type=jax.ShapeDtypeStruct((2, NPAD, HID), jnp.float32),
        mesh=plsc.VectorSubcoreMesh(core_axis_name="c", subcore_axis_name="s",
                                    num_cores=2, num_subcores=16),
        scratch_types=[
            pltpu.VMEM((EPT, HID), jnp.float32),
            pltpu.VMEM((PACK, SEG), jnp.int32),
            pltpu.VMEM((NCHUNK + 1, SCHUNK), jnp.int32),
            pltpu.VMEM_SHARED((NPAD, HID), jnp.float32),
            pltpu.SemaphoreType.DMA,
            pltpu.SemaphoreType.DMA,
            pltpu.SemaphoreType.DMA,
        ],
        compiler_params=pltpu.CompilerParams(
            use_tc_tiling_on_sc=False, needs_layout_passes=False),
    )


def _pool_head_body(part_ref, ids_ref, ne_ref, root_ref, bias_ref,
                    w4_ref, b4_ref, w5_ref, b5_ref, out_ref, sums, counts):
    i = pl.program_id(0)

    @pl.when(i == 0)
    def _init():
        sums[...] = jnp.zeros_like(sums)
        counts[...] = jnp.zeros_like(counts)

    c16 = (jnp.dot(ne_ref[...], root_ref[...],
                   preferred_element_type=jnp.float32) + bias_ref[...])
    c128 = jnp.concatenate([c16] * PACK, axis=1)
    x8 = jax.nn.relu(part_ref[0] + part_ref[1] + c128)
    for a in range(PACK):
        ids = ids_ref[a, :]
        onehot = (lax.broadcasted_iota(jnp.int32, (G, BNP), 0)
                  == ids[None, :]).astype(jnp.float32)
        xa = x8[:, a * HID:(a + 1) * HID]
        sums[...] += jnp.dot(onehot, xa,
                             preferred_element_type=jnp.float32)
        counts[...] += jnp.sum(onehot, axis=1, keepdims=True)

    @pl.when(i == pl.num_programs(0) - 1)
    def _final():
        pooled = sums[...] / jnp.maximum(counts[...], 1.0)
        h = jax.nn.relu(
            jnp.dot(pooled, w4_ref[...], preferred_element_type=jnp.float32)
            + b4_ref[...])
        out_ref[...] = (
            jnp.dot(h, w5_ref[...], preferred_element_type=jnp.float32)
            + b5_ref[...])


def _pool_head(partials, ids8, ne, root, bias2, w4, b42, w5, b52):
    return pl.pallas_call(
        _pool_head_body,
        grid=(NPK // BNP,),
        in_specs=[
            pl.BlockSpec((2, BNP, 128), lambda i: (0, i, 0)),
            pl.BlockSpec((PACK, BNP), lambda i: (0, i)),
            pl.BlockSpec((1, HID), lambda i: (0, 0)),
            pl.BlockSpec((HID, HID), lambda i: (0, 0)),
            pl.BlockSpec((1, HID), lambda i: (0, 0)),
            pl.BlockSpec((HID, 2 * HID), lambda i: (0, 0)),
            pl.BlockSpec((1, 2 * HID), lambda i: (0, 0)),
            pl.BlockSpec((2 * HID, OUT_DIM), lambda i: (0, 0)),
            pl.BlockSpec((1, OUT_DIM), lambda i: (0, 0)),
        ],
        out_specs=pl.BlockSpec((G, OUT_DIM), lambda i: (0, 0)),
        out_shape=jax.ShapeDtypeStruct((G, OUT_DIM), jnp.float32),
        scratch_shapes=[
            pltpu.VMEM((G, HID), jnp.float32),
            pltpu.VMEM((G, 1), jnp.float32),
        ],
    )(partials, ids8, ne, root, bias2, w4, b42, w5, b52)


def kernel(edge_attr, edge_index, batch, node_emb, w1, b1, w2, b2, w3, b3,
           root, bias, w4, b4, w5, b5):
    ne = node_emb[0]

    w3f = jnp.einsum("i,kio->ko", ne, w3.reshape(HID, EDGE_DIM, HID))
    b3f = ne @ b3.reshape(EDGE_DIM, HID)

    eat = edge_attr.T
    msg8 = _edge_mlp(eat, w1, b1[None, :], w2, b2[None, :], w3f,
                     b3f[None, :])
    msg = msg8.reshape(EP, HID)

    dst1 = jnp.concatenate(
        [edge_index[1], jnp.zeros((16,), dtype=jnp.int32)])
    zeros = jnp.zeros((NPT, HID), jnp.float32)
    partials = _build_scatter_add()(msg, dst1, zeros)

    ids8 = jnp.concatenate(
        [batch, jnp.full((NPAD - N,), G, dtype=jnp.int32)]
    ).reshape(NPK, PACK).T
    part8 = partials.reshape(2, NPK, 128)
    return _pool_head(part8, ids8, node_emb, root, bias[None, :],
                      w4, b4[None, :], w5, b5[None, :])


# speedup vs baseline: 24.1863x; 1.0367x over previous
import functools

import jax
import jax.numpy as jnp
from jax import lax
from jax.experimental import pallas as pl
from jax.experimental.pallas import tpu as pltpu
from jax.experimental.pallas import tpu_sc as plsc

N = 10000
E = 160000
EDGE_DIM = 16
HID = 16
OUT_DIM = 10
G = 64

PACK = 8
SUB = 10000
BE = PACK * SUB
NBLK = E // BE
EP = E
EPK = EP // PACK

NTILES = 32
EPT = EP // NTILES
RPT = EPT // PACK
TPB = NTILES // NBLK
SCHUNK = 128
NCHUNK = EPT // SCHUNK
TAIL = EPT - NCHUNK * SCHUNK
SEG = 648
NPAD = 10240
NPT = NPAD // 16

NPK = NPAD // PACK
BNP = NPK


def _edge_mlp_body(eat_ref, w1_ref, b1_ref, w2_ref, b2_ref, w3_ref, b3_ref,
                   out_ref):
    rb = lax.broadcasted_iota(jnp.int32, (128, 128), 0) // HID
    cb = lax.broadcasted_iota(jnp.int32, (128, 128), 1) // HID
    bmask = (rb == cb).astype(jnp.float32)
    w1b = jnp.tile(w1_ref[...], (PACK, PACK)) * bmask
    w2b = jnp.tile(w2_ref[...], (PACK, PACK)) * bmask
    w3b = jnp.tile(w3_ref[...], (PACK, PACK)) * bmask
    b1b = jnp.tile(b1_ref[...], (1, PACK))
    b2b = jnp.tile(b2_ref[...], (1, PACK))
    b3b = jnp.tile(b3_ref[...], (1, PACK))

    x = eat_ref[...]
    lhs = jnp.concatenate(
        [x[:, a * SUB:(a + 1) * SUB] for a in range(PACK)], axis=0)
    h = jax.nn.relu(
        lax.dot_general(lhs, w1b, (((0,), (0,)), ((), ())),
                        preferred_element_type=jnp.float32) + b1b)
    h = jax.nn.relu(
        jnp.dot(h, w2b, preferred_element_type=jnp.float32) + b2b)
    out_ref[...] = (
        jnp.dot(h, w3b, preferred_element_type=jnp.float32) + b3b)


def _edge_mlp(eat, w1, b1, w2, b2, w3f, b3f):
    wspec = pl.BlockSpec((HID, HID), lambda i: (0, 0))
    bspec = pl.BlockSpec((1, HID), lambda i: (0, 0))
    return pl.pallas_call(
        _edge_mlp_body,
        grid=(NBLK,),
        in_specs=[
            pl.BlockSpec((EDGE_DIM, BE), lambda i: (0, i)),
            wspec, bspec, wspec, bspec, wspec, bspec,
        ],
        out_specs=pl.BlockSpec((SUB, 128), lambda i: (i, 0)),
        out_shape=jax.ShapeDtypeStruct((EPK, 128), jnp.float32),
    )(eat, w1, b1, w2, b2, w3f, b3f)


def _scatter_body(msg_hbm, dst_hbm, zeros_hbm, out_hbm,
                  msg_v, stage_v, dst_v, acc, sem_m, sem_d, sem_s):
    c = lax.axis_index("c")
    s = lax.axis_index("s")
    wid = c * 16 + s
    blk = wid // TPB
    tloc = wid % TPB
    shift = (RPT * tloc) % 8

    msg_cp = pltpu.async_copy(msg_hbm.at[pl.ds(wid * EPT, EPT)], msg_v,
                              sem_m)

    dst_cps = []
    for a in range(PACK):
        base8 = (BE // 8) * blk + (SUB // 8) * a + (RPT * tloc) // 8
        dst_cps.append(pltpu.async_copy(
            dst_hbm.at[pl.ds(8 * base8, SEG)], stage_v.at[a], sem_d))

    pltpu.sync_copy(zeros_hbm, acc.at[pl.ds(s * NPT, NPT)])
    plsc.subcore_barrier()
    for cp in dst_cps:
        cp.wait()

    io = lax.iota(jnp.int32, 16)
    for a in range(PACK):
        row = jnp.full((16,), a, jnp.int32)
        def reorder(g, _):
            rho = 16 * g + io
            mask = rho < RPT
            vals = plsc.load_gather(stage_v, [row, shift + rho], mask=mask)
            m = 8 * rho + a
            plsc.store_scatter(dst_v, [m >> 7, m & (SCHUNK - 1)], vals,
                               mask=mask)
            return 0
        lax.fori_loop(0, 40, reorder, 0)

    msg_cp.wait()
    cps = [pltpu.async_copy(msg_v.at[pl.ds(j * SCHUNK, SCHUNK)],
                            acc.at[dst_v.at[j]], sem_s, add=True)
           for j in range(NCHUNK)]
    cps.append(pltpu.async_copy(
        msg_v.at[pl.ds(NCHUNK * SCHUNK, TAIL)],
        acc.at[dst_v.at[NCHUNK, pl.ds(0, TAIL)]], sem_s, add=True))
    for cp in cps:
        cp.wait()
    plsc.subcore_barrier()

    pltpu.sync_copy(acc.at[pl.ds(s * NPT, NPT)],
                    out_hbm.at[c, pl.ds(s * NPT, NPT)])


@functools.cache
def _build_scatter_add():
    return pl.kernel(
        _scatter_body,
        out_type=jax.ShapeDtypeStruct((2, NPAD, HID), jnp.float32),
        mesh=plsc.VectorSubcoreMesh(core_axis_name="c", subcore_axis_name="s",
                                    num_cores=2, num_subcores=16),
        scratch_types=[
            pltpu.VMEM((EPT, HID), jnp.float32),
            pltpu.VMEM((PACK, SEG), jnp.int32),
            pltpu.VMEM((NCHUNK + 1, SCHUNK), jnp.int32),
            pltpu.VMEM_SHARED((NPAD, HID), jnp.float32),
            pltpu.SemaphoreType.DMA,
            pltpu.SemaphoreType.DMA,
            pltpu.SemaphoreType.DMA,
        ],
        compiler_params=pltpu.CompilerParams(
            use_tc_tiling_on_sc=False, needs_layout_passes=False),
    )


def _pool_head_body(part_ref, ids_ref, ne_ref, root_ref, bias_ref,
                    w4_ref, b4_ref, w5_ref, b5_ref, out_ref, sums, counts):
    i = pl.program_id(0)

    @pl.when(i == 0)
    def _init():
        sums[...] = jnp.zeros_like(sums)
        counts[...] = jnp.zeros_like(counts)

    c16 = (jnp.dot(ne_ref[...], root_ref[...],
                   preferred_element_type=jnp.float32) + bias_ref[...])
    c128 = jnp.concatenate([c16] * PACK, axis=1)
    x8 = jax.nn.relu(part_ref[0] + part_ref[1] + c128)
    for a in range(PACK):
        ids = ids_ref[a, :]
        onehot = (lax.broadcasted_iota(jnp.int32, (G, BNP), 0)
                  == ids[None, :]).astype(jnp.float32)
        xa = x8[:, a * HID:(a + 1) * HID]
        sums[...] += jnp.dot(onehot, xa,
                             preferred_element_type=jnp.float32)
        counts[...] += jnp.sum(onehot, axis=1, keepdims=True)

    @pl.when(i == pl.num_programs(0) - 1)
    def _final():
        pooled = sums[...] / jnp.maximum(counts[...], 1.0)
        h = jax.nn.relu(
            jnp.dot(pooled, w4_ref[...], preferred_element_type=jnp.float32)
            + b4_ref[...])
        out_ref[...] = (
            jnp.dot(h, w5_ref[...], preferred_element_type=jnp.float32)
            + b5_ref[...])


def _pool_head(partials, ids8, ne, root, bias2, w4, b42, w5, b52):
    return pl.pallas_call(
        _pool_head_body,
        grid=(NPK // BNP,),
        in_specs=[
            pl.BlockSpec((2, BNP, 128), lambda i: (0, i, 0)),
            pl.BlockSpec((PACK, BNP), lambda i: (0, i)),
            pl.BlockSpec((1, HID), lambda i: (0, 0)),
            pl.BlockSpec((HID, HID), lambda i: (0, 0)),
            pl.BlockSpec((1, HID), lambda i: (0, 0)),
            pl.BlockSpec((HID, 2 * HID), lambda i: (0, 0)),
            pl.BlockSpec((1, 2 * HID), lambda i: (0, 0)),
            pl.BlockSpec((2 * HID, OUT_DIM), lambda i: (0, 0)),
            pl.BlockSpec((1, OUT_DIM), lambda i: (0, 0)),
        ],
        out_specs=pl.BlockSpec((G, OUT_DIM), lambda i: (0, 0)),
        out_shape=jax.ShapeDtypeStruct((G, OUT_DIM), jnp.float32),
        scratch_shapes=[
            pltpu.VMEM((G, HID), jnp.float32),
            pltpu.VMEM((G, 1), jnp.float32),
        ],
    )(partials, ids8, ne, root, bias2, w4, b42, w5, b52)


def kernel(edge_attr, edge_index, batch, node_emb, w1, b1, w2, b2, w3, b3,
           root, bias, w4, b4, w5, b5):
    ne = node_emb[0]

    w3f = jnp.einsum("i,kio->ko", ne, w3.reshape(HID, EDGE_DIM, HID))
    b3f = ne @ b3.reshape(EDGE_DIM, HID)

    eat = edge_attr.T
    msg8 = _edge_mlp(eat, w1, b1[None, :], w2, b2[None, :], w3f,
                     b3f[None, :])
    msg = msg8.reshape(EP, HID)

    dst1 = jnp.concatenate(
        [edge_index[1], jnp.zeros((16,), dtype=jnp.int32)])
    zeros = jnp.zeros((NPT, HID), jnp.float32)
    partials = _build_scatter_add()(msg, dst1, zeros)

    ids8 = jnp.concatenate(
        [batch, jnp.full((NPAD - N,), G, dtype=jnp.int32)]
    ).reshape(NPK, PACK).T
    part8 = partials.reshape(2, NPK, 128)
    return _pool_head(part8, ids8, node_emb, root, bias[None, :],
                      w4, b4[None, :], w5, b5[None, :])
